# baseline (speedup 1.0000x reference)
import functools

import jax
import jax.numpy as jnp
from jax import lax
from jax.experimental import pallas as pl
from jax.experimental.pallas import tpu as pltpu
from jax.experimental.pallas import tpu_sc as plsc

VOCAB = 1000
EMBED_DIM = 64
B = 4096
L = 200

N = B * L
NUM_WORKERS = 32
PER_WORKER = N // NUM_WORKERS
LANES = 16
SUPER = 256
GROUPS_PER_SUPER = SUPER // LANES
NUM_SUPER = PER_WORKER // SUPER


_mesh = plsc.VectorSubcoreMesh(core_axis_name="c", subcore_axis_name="s")


@functools.partial(
    pl.kernel,
    mesh=_mesh,
    out_type=jax.ShapeDtypeStruct((N, EMBED_DIM), jnp.float32),
    scratch_types=[
        pltpu.VMEM((VOCAB, EMBED_DIM), jnp.float32),
        pltpu.VMEM((PER_WORKER,), jnp.int32),
        pltpu.VMEM((SUPER, EMBED_DIM), jnp.float32),
        pltpu.VMEM((SUPER, EMBED_DIM), jnp.float32),
        pltpu.SemaphoreType.DMA,
        pltpu.SemaphoreType.DMA,
    ],
    compiler_params=pltpu.CompilerParams(
        use_tc_tiling_on_sc=False, needs_layout_passes=False),
)
def _embed_sc(news_hbm, table_hbm, out_hbm, table_v, idx_v, buf0, buf1,
              ssem0, ssem1):
    wid = lax.axis_index("s") * 2 + lax.axis_index("c")
    base = wid * PER_WORKER

    pltpu.sync_copy(table_hbm, table_v)
    pltpu.sync_copy(news_hbm.at[wid], idx_v)

    bufs = (buf0, buf1)
    ssems = (ssem0, ssem1)
    lane = lax.iota(jnp.int32, LANES)

    def wait_store(buf, ssem):
        pltpu.make_async_copy(buf, out_hbm.at[pl.ds(base, SUPER)], ssem).wait()

    def fill_super(s, buf):
        def group(g, _):
            idx16 = idx_v[pl.ds(s * SUPER + g * LANES, LANES)]
            pos16 = lane + g * LANES
            for j in range(EMBED_DIM):
                col = jnp.full((LANES,), j, jnp.int32)
                vals = plsc.load_gather(table_v, [idx16, col])
                plsc.store_scatter(buf, [pos16, col], vals)
            return 0

        lax.fori_loop(0, GROUPS_PER_SUPER, group, 0)

    def body(i, _):
        for par in range(2):
            s = i * 2 + par
            b = par

            @pl.when(s >= 2)
            def _():
                wait_store(bufs[b], ssems[b])

            fill_super(s, bufs[b])
            pltpu.async_copy(
                bufs[b], out_hbm.at[pl.ds(base + s * SUPER, SUPER)], ssems[b])
        return 0

    lax.fori_loop(0, NUM_SUPER // 2, body, 0)

    wait_store(buf0, ssem0)
    wait_store(buf1, ssem1)


def kernel(news, news_mask, table):
    del news_mask
    idx = news.reshape(NUM_WORKERS, PER_WORKER)
    out = _embed_sc(idx, table)
    return out.reshape(B, L, EMBED_DIM)


# speedup vs baseline: 2.6633x; 2.6633x over previous
import functools

import jax
import jax.numpy as jnp
from jax import lax
from jax.experimental import pallas as pl
from jax.experimental.pallas import tpu as pltpu
from jax.experimental.pallas import tpu_sc as plsc

VOCAB = 1000
EMBED_DIM = 64
B = 4096
L = 200

N = B * L
NUM_WORKERS = 32
PER_WORKER = N // NUM_WORKERS
LANES = 16
SUPER = 256
STREAM = 192
COMPUTE_GROUPS = (SUPER - STREAM) // LANES
NUM_SUPER = PER_WORKER // SUPER


_mesh = plsc.VectorSubcoreMesh(core_axis_name="c", subcore_axis_name="s")


@functools.partial(
    pl.kernel,
    mesh=_mesh,
    out_type=jax.ShapeDtypeStruct((N, EMBED_DIM), jnp.float32),
    scratch_types=[
        pltpu.VMEM_SHARED((VOCAB, EMBED_DIM), jnp.float32),
        pltpu.VMEM((VOCAB, EMBED_DIM), jnp.float32),
        pltpu.VMEM((NUM_SUPER, SUPER), jnp.int32),
        pltpu.VMEM((SUPER, EMBED_DIM), jnp.float32),
        pltpu.VMEM((SUPER, EMBED_DIM), jnp.float32),
        pltpu.SemaphoreType.DMA,
        pltpu.SemaphoreType.DMA,
        pltpu.SemaphoreType.DMA,
        pltpu.SemaphoreType.DMA,
    ],
    compiler_params=pltpu.CompilerParams(
        use_tc_tiling_on_sc=False, needs_layout_passes=False),
)
def _embed_sc(news_hbm, table_hbm, out_hbm, table_sh, table_v, idx_v,
              buf0, buf1, gsem0, gsem1, ssem0, ssem1):
    sid = lax.axis_index("s")
    wid = sid * 2 + lax.axis_index("c")
    base = wid * PER_WORKER

    @pl.when(sid == 0)
    def _():
        pltpu.sync_copy(table_hbm, table_sh)

    pltpu.sync_copy(news_hbm.at[wid], idx_v)
    plsc.subcore_barrier()
    pltpu.sync_copy(table_sh, table_v)

    bufs = (buf0, buf1)
    gsems = (gsem0, gsem1)
    ssems = (ssem0, ssem1)
    lane = lax.iota(jnp.int32, LANES)

    def wait_store(buf, ssem):
        pltpu.make_async_copy(buf, out_hbm.at[pl.ds(base, SUPER)], ssem).wait()

    def body(i, _):
        for par in range(2):
            s = i * 2 + par
            b = par
            buf = bufs[b]

            @pl.when(s >= 2)
            def _():
                wait_store(buf, ssems[b])

            pltpu.async_copy(
                table_sh.at[idx_v.at[s, pl.ds(0, STREAM)]],
                buf.at[pl.ds(0, STREAM)],
                gsems[b],
            )

            @plsc.parallel_loop(0, COMPUTE_GROUPS)
            def _(g):
                off = STREAM + g * LANES
                idx16 = idx_v[s, pl.ds(off, LANES)]
                pos16 = lane + off
                for j in range(EMBED_DIM):
                    col = jnp.full((LANES,), j, jnp.int32)
                    vals = plsc.load_gather(table_v, [idx16, col])
                    plsc.store_scatter(buf, [pos16, col], vals)

            pltpu.make_async_copy(
                table_sh.at[idx_v.at[s, pl.ds(0, STREAM)]],
                buf.at[pl.ds(0, STREAM)],
                gsems[b],
            ).wait()
            pltpu.async_copy(
                buf, out_hbm.at[pl.ds(base + s * SUPER, SUPER)], ssems[b])
        return 0

    lax.fori_loop(0, NUM_SUPER // 2, body, 0)

    wait_store(buf0, ssem0)
    wait_store(buf1, ssem1)


def kernel(news, news_mask, table):
    del news_mask
    idx = news.reshape(NUM_WORKERS, NUM_SUPER, SUPER)
    out = _embed_sc(idx, table)
    return out.reshape(B, L, EMBED_DIM)


# speedup vs baseline: 4.0067x; 1.5044x over previous
import functools

import jax
import jax.numpy as jnp
from jax import lax
from jax.experimental import pallas as pl
from jax.experimental.pallas import tpu as pltpu
from jax.experimental.pallas import tpu_sc as plsc

VOCAB = 1000
EMBED_DIM = 64
B = 4096
L = 200

N = B * L
NUM_WORKERS = 32
PER_WORKER = N // NUM_WORKERS
GROUP = 128
SUPER = 512
CHUNKS = SUPER // GROUP
NUM_SUPER = PER_WORKER // SUPER


_mesh = plsc.VectorSubcoreMesh(core_axis_name="c", subcore_axis_name="s")


@functools.partial(
    pl.kernel,
    mesh=_mesh,
    out_type=jax.ShapeDtypeStruct((N, EMBED_DIM), jnp.float32),
    scratch_types=[
        pltpu.VMEM_SHARED((VOCAB, EMBED_DIM), jnp.float32),
        pltpu.VMEM((NUM_SUPER, SUPER), jnp.int32),
        pltpu.VMEM((SUPER, EMBED_DIM), jnp.float32),
        pltpu.VMEM((SUPER, EMBED_DIM), jnp.float32),
        [pltpu.SemaphoreType.DMA] * CHUNKS,
        [pltpu.SemaphoreType.DMA] * CHUNKS,
        pltpu.SemaphoreType.DMA,
        pltpu.SemaphoreType.DMA,
    ],
    compiler_params=pltpu.CompilerParams(use_tc_tiling_on_sc=False),
)
def _embed_sc(news_hbm, table_hbm, out_hbm, table_sh, idx_v, buf0, buf1,
              gsems0, gsems1, ssem0, ssem1):
    sid = lax.axis_index("s")
    wid = sid * 2 + lax.axis_index("c")
    base = wid * PER_WORKER

    @pl.when(sid == 0)
    def _():
        pltpu.sync_copy(table_hbm, table_sh)

    pltpu.sync_copy(news_hbm.at[wid], idx_v)
    plsc.subcore_barrier()

    bufs = (buf0, buf1)
    gsems = (gsems0, gsems1)
    ssems = (ssem0, ssem1)

    def start_gathers(s, buf, gs):
        for c in range(CHUNKS):
            pltpu.async_copy(
                table_sh.at[idx_v.at[s, pl.ds(c * GROUP, GROUP)]],
                buf.at[pl.ds(c * GROUP, GROUP)],
                gs[c],
            )

    def wait_gathers(s, buf, gs):
        for c in range(CHUNKS):
            pltpu.make_async_copy(
                table_sh.at[idx_v.at[s, pl.ds(c * GROUP, GROUP)]],
                buf.at[pl.ds(c * GROUP, GROUP)],
                gs[c],
            ).wait()

    def wait_store(buf, ssem):
        pltpu.make_async_copy(buf, out_hbm.at[pl.ds(base, SUPER)], ssem).wait()

    start_gathers(0, buf0, gsems0)

    def body(i, _):
        for par in range(2):
            s = i * 2 + par
            b = par
            nb = 1 - par

            @pl.when(s >= 1)
            def _():
                wait_store(bufs[nb], ssems[nb])

            @pl.when(s + 1 < NUM_SUPER)
            def _():
                start_gathers(s + 1, bufs[nb], gsems[nb])

            wait_gathers(s, bufs[b], gsems[b])
            pltpu.async_copy(
                bufs[b], out_hbm.at[pl.ds(base + s * SUPER, SUPER)], ssems[b])
        return 0

    lax.fori_loop(0, NUM_SUPER // 2, body, 0)

    wait_store(buf1, ssem1)


def kernel(news, news_mask, table):
    del news_mask
    idx = news.reshape(NUM_WORKERS, NUM_SUPER, SUPER)
    out = _embed_sc(idx, table)
    return out.reshape(B, L, EMBED_DIM)
